# baseline (speedup 1.0000x reference)
import numpy as np

import jax
import jax.numpy as jnp
from jax import lax
from jax.experimental import pallas as pl
from jax.experimental.pallas import tpu as pltpu
from jax.experimental.pallas import tpu_sc as plsc

N = 10000
E = 320000
D = 128
NC = 2
NS = 16
NW = NC * NS
CH = 128
NCH = 79
EPAD = NW * NCH * CH
ACC_N = 10240
RPT = ACC_N // NS

_PERM = np.empty((D,), dtype=np.int32)
for _g in range(D // 32):
    for _i in range(16):
        _PERM[32 * _g + 2 * _i] = 32 * _g + _i
        _PERM[32 * _g + 2 * _i + 1] = 32 * _g + 16 + _i

_MESH = plsc.VectorSubcoreMesh(core_axis_name="c", subcore_axis_name="s")


def _spmm_body(y_hbm, rc_hbm, val_hbm, out_hbm,
               acc, rcb, valb, rowc, colc, gb, sbuf, gs0):
    c = lax.axis_index("c")
    s = lax.axis_index("s")
    wid = c * NS + s

    pltpu.async_copy(rc_hbm.at[wid], rcb, gs0)
    pltpu.async_copy(val_hbm.at[wid], valb, gs0)

    zero = jnp.zeros((16,), jnp.float32)

    def zb(i, carry):
        for j in range(8):
            sbuf[i, pl.ds(j * 16, 16)] = zero
        return carry

    lax.fori_loop(0, CH, zb, 0)
    for t in range(RPT // CH):
        pltpu.sync_copy(sbuf, acc.at[pl.ds(s * RPT + t * CH, CH)])
    pltpu.make_async_copy(rc_hbm.at[wid], rcb, gs0).wait()
    pltpu.make_async_copy(val_hbm.at[wid], valb, gs0).wait()
    plsc.subcore_barrier()

    def chunk(k, carry):
        def unpack_idx(g, inner):
            rcv = rcb[k, pl.ds(g * 16, 16)]
            colc[pl.ds(g * 16, 16)] = rcv & jnp.int32(0xFFFF)
            rowc[pl.ds(g * 16, 16)] = lax.shift_right_logical(
                rcv, jnp.int32(16))
            return inner

        lax.fori_loop(0, CH // 16, unpack_idx, 0)

        pltpu.async_copy(y_hbm.at[colc], gb, gs0)
        pltpu.make_async_copy(y_hbm.at[pl.ds(0, CH)], gb, gs0).wait()

        def edge_group(g, inner):
            vvec = valb[k, pl.ds(g * 16, 16)]
            for l in range(16):
                v = vvec[l]
                e = g * 16 + l
                for j in range(D // 32):
                    u = gb[e, pl.ds(j * 16, 16)]
                    lo = plsc.bitcast(u << jnp.int32(16), jnp.float32)
                    hi = plsc.bitcast(u & jnp.int32(-65536), jnp.float32)
                    sbuf[e, pl.ds(j * 32, 16)] = lo * v
                    sbuf[e, pl.ds(j * 32 + 16, 16)] = hi * v
            return inner

        lax.fori_loop(0, CH // 16, edge_group, 0)
        pltpu.sync_copy(sbuf, acc.at[rowc], add=True)
        return carry

    lax.fori_loop(0, NCH, chunk, 0)
    plsc.subcore_barrier()

    pltpu.sync_copy(acc.at[pl.ds(s * RPT, RPT)],
                    out_hbm.at[c, pl.ds(s * RPT, RPT)])


_spmm = pl.kernel(
    _spmm_body,
    out_type=jax.ShapeDtypeStruct((NC, ACC_N, D), jnp.float32),
    mesh=_MESH,
    compiler_params=pltpu.CompilerParams(use_tc_tiling_on_sc=False,
                                         needs_layout_passes=False),
    scratch_types=[
        pltpu.VMEM_SHARED((ACC_N, D), jnp.float32),
        pltpu.VMEM((NCH, CH), jnp.int32),
        pltpu.VMEM((NCH, CH), jnp.float32),
        pltpu.VMEM((CH,), jnp.int32),
        pltpu.VMEM((CH,), jnp.int32),
        pltpu.VMEM((CH, D // 2), jnp.int32),
        pltpu.VMEM((CH, D), jnp.float32),
        pltpu.SemaphoreType.DMA,
    ],
)


ROWS_BLK = 1000
GRID = N // ROWS_BLK


def _gemm0_body(x_ref, w_ref, y_ref):
    y_ref[...] = jnp.dot(x_ref[...], w_ref[...].T,
                         preferred_element_type=jnp.float32
                         ).astype(jnp.bfloat16)


_gemm0 = pl.pallas_call(
    _gemm0_body,
    grid=(GRID,),
    in_specs=[
        pl.BlockSpec((ROWS_BLK, D), lambda i: (i, 0)),
        pl.BlockSpec((D, D), lambda i: (0, 0)),
    ],
    out_specs=pl.BlockSpec((ROWS_BLK, D), lambda i: (i, 0)),
    out_shape=jax.ShapeDtypeStruct((N, D), jnp.bfloat16),
)


def _gemm_mid_body(p_ref, w_ref, x_ref, y_ref):
    x = p_ref[0] + p_ref[1]
    x_ref[...] = x
    y_ref[...] = jnp.dot(x, w_ref[...].T,
                         preferred_element_type=jnp.float32
                         ).astype(jnp.bfloat16)


_gemm_mid = pl.pallas_call(
    _gemm_mid_body,
    grid=(GRID,),
    in_specs=[
        pl.BlockSpec((NC, ROWS_BLK, D), lambda i: (0, i, 0)),
        pl.BlockSpec((D, D), lambda i: (0, 0)),
    ],
    out_specs=[
        pl.BlockSpec((ROWS_BLK, D), lambda i: (i, 0)),
        pl.BlockSpec((ROWS_BLK, D), lambda i: (i, 0)),
    ],
    out_shape=[
        jax.ShapeDtypeStruct((N, D), jnp.float32),
        jax.ShapeDtypeStruct((N, D), jnp.bfloat16),
    ],
)


def _normed(x):
    nrm = jnp.sqrt(jnp.sum(x * x, axis=-1, keepdims=True))
    return x / jnp.maximum(nrm, 1e-12)


def _final_body(x0_ref, x1_ref, x2_ref, p_ref, o_ref):
    x3 = p_ref[0] + p_ref[1]
    o_ref[...] = 0.25 * (x0_ref[...] + _normed(x1_ref[...])
                         + _normed(x2_ref[...]) + _normed(x3))


_final = pl.pallas_call(
    _final_body,
    grid=(GRID,),
    in_specs=[
        pl.BlockSpec((ROWS_BLK, D), lambda i: (i, 0)),
        pl.BlockSpec((ROWS_BLK, D), lambda i: (i, 0)),
        pl.BlockSpec((ROWS_BLK, D), lambda i: (i, 0)),
        pl.BlockSpec((NC, ROWS_BLK, D), lambda i: (0, i, 0)),
    ],
    out_specs=pl.BlockSpec((ROWS_BLK, D), lambda i: (i, 0)),
    out_shape=jax.ShapeDtypeStruct((N, D), jnp.float32),
)


def _pack_pairs(y16):
    return jax.lax.bitcast_convert_type(
        y16.reshape(N, D // 2, 2), jnp.int32)


def kernel(adjacency_row, adjacency_col, adjacency_values, embedding, weights):
    pad = EPAD - E
    rc3 = jnp.concatenate(
        [(adjacency_row << 16) | adjacency_col,
         jnp.zeros((pad,), jnp.int32)]).reshape(NW, NCH, CH)
    val3 = jnp.concatenate(
        [adjacency_values, jnp.zeros((pad,), jnp.float32)]).reshape(
            NW, NCH, CH)
    perm = jnp.asarray(_PERM)
    wp = weights[:, perm, :]

    y0 = _gemm0(embedding, wp[0])
    p1 = _spmm(_pack_pairs(y0), rc3, val3)
    x1, y1 = _gemm_mid(p1, wp[1])
    p2 = _spmm(_pack_pairs(y1), rc3, val3)
    x2, y2 = _gemm_mid(p2, wp[2])
    p3 = _spmm(_pack_pairs(y2), rc3, val3)
    return _final(embedding, x1, x2, p3)


# speedup vs baseline: 1.0298x; 1.0298x over previous
import numpy as np

import jax
import jax.numpy as jnp
from jax import lax
from jax.experimental import pallas as pl
from jax.experimental.pallas import tpu as pltpu
from jax.experimental.pallas import tpu_sc as plsc

N = 10000
E = 320000
D = 128
NC = 2
NS = 16
NW = NC * NS
CH = 96
NCH = 105
EPAD = NW * NCH * CH
ACC_N = 10240
RPT = ACC_N // NS

_PERM = np.empty((D,), dtype=np.int32)
for _g in range(D // 32):
    for _i in range(16):
        _PERM[32 * _g + 2 * _i] = 32 * _g + _i
        _PERM[32 * _g + 2 * _i + 1] = 32 * _g + 16 + _i

_MESH = plsc.VectorSubcoreMesh(core_axis_name="c", subcore_axis_name="s")


def _spmm_body(y_hbm, row_hbm, col_hbm, val_hbm, out_hbm,
               acc, rowb, colb, valb, gb, sbuf, gs0):
    c = lax.axis_index("c")
    s = lax.axis_index("s")
    wid = c * NS + s

    pltpu.async_copy(row_hbm.at[wid], rowb, gs0)
    pltpu.async_copy(col_hbm.at[wid], colb, gs0)
    pltpu.async_copy(val_hbm.at[wid], valb, gs0)

    zero = jnp.zeros((16,), jnp.float32)

    def zb(i, carry):
        for j in range(8):
            sbuf[i, pl.ds(j * 16, 16)] = zero
        return carry

    lax.fori_loop(0, CH, zb, 0)
    for t in range(-(-RPT // CH)):
        off = min(t * CH, RPT - CH)
        pltpu.sync_copy(sbuf, acc.at[pl.ds(s * RPT + off, CH)])
    pltpu.make_async_copy(row_hbm.at[wid], rowb, gs0).wait()
    pltpu.make_async_copy(col_hbm.at[wid], colb, gs0).wait()
    pltpu.make_async_copy(val_hbm.at[wid], valb, gs0).wait()
    plsc.subcore_barrier()

    def chunk(k, carry):
        pltpu.async_copy(y_hbm.at[colb.at[k]], gb, gs0)
        pltpu.make_async_copy(y_hbm.at[pl.ds(0, CH)], gb, gs0).wait()

        def edge_group(g, inner):
            vvec = valb[k, pl.ds(g * 16, 16)]
            for l in range(16):
                v = vvec[l]
                e = g * 16 + l
                for j in range(D // 32):
                    u = gb[e, pl.ds(j * 16, 16)]
                    lo = plsc.bitcast(u << jnp.int32(16), jnp.float32)
                    hi = plsc.bitcast(u & jnp.int32(-65536), jnp.float32)
                    sbuf[e, pl.ds(j * 32, 16)] = lo * v
                    sbuf[e, pl.ds(j * 32 + 16, 16)] = hi * v
            return inner

        lax.fori_loop(0, CH // 16, edge_group, 0)
        pltpu.sync_copy(sbuf, acc.at[rowb.at[k]], add=True)
        return carry

    lax.fori_loop(0, NCH, chunk, 0)
    plsc.subcore_barrier()

    pltpu.sync_copy(acc.at[pl.ds(s * RPT, RPT)],
                    out_hbm.at[c, pl.ds(s * RPT, RPT)])


_spmm = pl.kernel(
    _spmm_body,
    out_type=jax.ShapeDtypeStruct((NC, ACC_N, D), jnp.float32),
    mesh=_MESH,
    compiler_params=pltpu.CompilerParams(use_tc_tiling_on_sc=False,
                                         needs_layout_passes=False),
    scratch_types=[
        pltpu.VMEM_SHARED((ACC_N, D), jnp.float32),
        pltpu.VMEM((NCH, CH), jnp.int32),
        pltpu.VMEM((NCH, CH), jnp.int32),
        pltpu.VMEM((NCH, CH), jnp.float32),
        pltpu.VMEM((CH, D // 2), jnp.int32),
        pltpu.VMEM((CH, D), jnp.float32),
        pltpu.SemaphoreType.DMA,
    ],
)


ROWS_BLK = 1000
GRID = N // ROWS_BLK


def _gemm0_body(x_ref, w_ref, y_ref):
    y_ref[...] = jnp.dot(x_ref[...], w_ref[...].T,
                         preferred_element_type=jnp.float32
                         ).astype(jnp.bfloat16)


_gemm0 = pl.pallas_call(
    _gemm0_body,
    grid=(GRID,),
    in_specs=[
        pl.BlockSpec((ROWS_BLK, D), lambda i: (i, 0)),
        pl.BlockSpec((D, D), lambda i: (0, 0)),
    ],
    out_specs=pl.BlockSpec((ROWS_BLK, D), lambda i: (i, 0)),
    out_shape=jax.ShapeDtypeStruct((N, D), jnp.bfloat16),
)


def _gemm_mid_body(p_ref, w_ref, x_ref, y_ref):
    x = p_ref[0] + p_ref[1]
    x_ref[...] = x
    y_ref[...] = jnp.dot(x, w_ref[...].T,
                         preferred_element_type=jnp.float32
                         ).astype(jnp.bfloat16)


_gemm_mid = pl.pallas_call(
    _gemm_mid_body,
    grid=(GRID,),
    in_specs=[
        pl.BlockSpec((NC, ROWS_BLK, D), lambda i: (0, i, 0)),
        pl.BlockSpec((D, D), lambda i: (0, 0)),
    ],
    out_specs=[
        pl.BlockSpec((ROWS_BLK, D), lambda i: (i, 0)),
        pl.BlockSpec((ROWS_BLK, D), lambda i: (i, 0)),
    ],
    out_shape=[
        jax.ShapeDtypeStruct((N, D), jnp.float32),
        jax.ShapeDtypeStruct((N, D), jnp.bfloat16),
    ],
)


def _normed(x):
    nrm = jnp.sqrt(jnp.sum(x * x, axis=-1, keepdims=True))
    return x / jnp.maximum(nrm, 1e-12)


def _final_body(x0_ref, x1_ref, x2_ref, p_ref, o_ref):
    x3 = p_ref[0] + p_ref[1]
    o_ref[...] = 0.25 * (x0_ref[...] + _normed(x1_ref[...])
                         + _normed(x2_ref[...]) + _normed(x3))


_final = pl.pallas_call(
    _final_body,
    grid=(GRID,),
    in_specs=[
        pl.BlockSpec((ROWS_BLK, D), lambda i: (i, 0)),
        pl.BlockSpec((ROWS_BLK, D), lambda i: (i, 0)),
        pl.BlockSpec((ROWS_BLK, D), lambda i: (i, 0)),
        pl.BlockSpec((NC, ROWS_BLK, D), lambda i: (0, i, 0)),
    ],
    out_specs=pl.BlockSpec((ROWS_BLK, D), lambda i: (i, 0)),
    out_shape=jax.ShapeDtypeStruct((N, D), jnp.float32),
)


def _pack_pairs(y16):
    return jax.lax.bitcast_convert_type(
        y16.reshape(N, D // 2, 2), jnp.int32)


def kernel(adjacency_row, adjacency_col, adjacency_values, embedding, weights):
    pad = EPAD - E
    row3 = jnp.concatenate(
        [adjacency_row, jnp.zeros((pad,), jnp.int32)]).reshape(NW, NCH, CH)
    col3 = jnp.concatenate(
        [adjacency_col, jnp.zeros((pad,), jnp.int32)]).reshape(NW, NCH, CH)
    val3 = jnp.concatenate(
        [adjacency_values, jnp.zeros((pad,), jnp.float32)]).reshape(
            NW, NCH, CH)
    perm = jnp.asarray(_PERM)
    wp = weights[:, perm, :]

    y0 = _gemm0(embedding, wp[0])
    p1 = _spmm(_pack_pairs(y0), row3, col3, val3)
    x1, y1 = _gemm_mid(p1, wp[1])
    p2 = _spmm(_pack_pairs(y1), row3, col3, val3)
    x2, y2 = _gemm_mid(p2, wp[2])
    p3 = _spmm(_pack_pairs(y2), row3, col3, val3)
    return _final(embedding, x1, x2, p3)


# speedup vs baseline: 1.4715x; 1.4288x over previous
import numpy as np

import jax
import jax.numpy as jnp
from jax import lax
from jax.experimental import pallas as pl
from jax.experimental.pallas import tpu as pltpu
from jax.experimental.pallas import tpu_sc as plsc

N = 10000
E = 320000
D = 128
NC = 2
NS = 16
NW = NC * NS
CH = 96
NCH = 105
EPAD = NW * NCH * CH
ACC_N = 10240
RPT = ACC_N // NS

_PERM = np.empty((D,), dtype=np.int32)
for _g in range(D // 32):
    for _i in range(16):
        _PERM[32 * _g + 2 * _i] = 32 * _g + _i
        _PERM[32 * _g + 2 * _i + 1] = 32 * _g + 16 + _i

_MESH = plsc.VectorSubcoreMesh(core_axis_name="c", subcore_axis_name="s")


def _spmm_body(y_hbm, row_hbm, col_hbm, val_hbm, out_hbm,
               acc, rowb, colb, valb, gb, sbuf, gs0):
    c = lax.axis_index("c")
    s = lax.axis_index("s")
    wid = c * NS + s

    pltpu.async_copy(row_hbm.at[wid], rowb, gs0)
    pltpu.async_copy(col_hbm.at[wid], colb, gs0)
    pltpu.async_copy(val_hbm.at[wid], valb, gs0)

    zero = jnp.zeros((16,), jnp.float32)

    def zb(i, carry):
        for j in range(8):
            sbuf[i, pl.ds(j * 16, 16)] = zero
        return carry

    lax.fori_loop(0, CH, zb, 0)
    for t in range(-(-RPT // CH)):
        off = min(t * CH, RPT - CH)
        pltpu.sync_copy(sbuf, acc.at[pl.ds(s * RPT + off, CH)])
    pltpu.make_async_copy(row_hbm.at[wid], rowb, gs0).wait()
    pltpu.make_async_copy(col_hbm.at[wid], colb, gs0).wait()
    pltpu.make_async_copy(val_hbm.at[wid], valb, gs0).wait()
    plsc.subcore_barrier()

    def chunk(k, carry):
        pltpu.async_copy(y_hbm.at[colb.at[k]], gb, gs0)
        pltpu.make_async_copy(y_hbm.at[pl.ds(0, CH)], gb, gs0).wait()

        @plsc.parallel_loop(0, CH // 16)
        def edge_group(g):
            vvec = valb[k, pl.ds(g * 16, 16)]
            for l in range(16):
                v = vvec[l]
                e = g * 16 + l
                for j in range(D // 32):
                    u = gb[e, pl.ds(j * 16, 16)]
                    lo = plsc.bitcast(u << jnp.int32(16), jnp.float32)
                    hi = plsc.bitcast(u & jnp.int32(-65536), jnp.float32)
                    sbuf[e, pl.ds(j * 32, 16)] = lo * v
                    sbuf[e, pl.ds(j * 32 + 16, 16)] = hi * v
        pltpu.sync_copy(sbuf, acc.at[rowb.at[k]], add=True)
        return carry

    lax.fori_loop(0, NCH, chunk, 0)
    plsc.subcore_barrier()

    pltpu.sync_copy(acc.at[pl.ds(s * RPT, RPT)],
                    out_hbm.at[c, pl.ds(s * RPT, RPT)])


_spmm = pl.kernel(
    _spmm_body,
    out_type=jax.ShapeDtypeStruct((NC, ACC_N, D), jnp.float32),
    mesh=_MESH,
    compiler_params=pltpu.CompilerParams(use_tc_tiling_on_sc=False,
                                         needs_layout_passes=False),
    scratch_types=[
        pltpu.VMEM_SHARED((ACC_N, D), jnp.float32),
        pltpu.VMEM((NCH, CH), jnp.int32),
        pltpu.VMEM((NCH, CH), jnp.int32),
        pltpu.VMEM((NCH, CH), jnp.float32),
        pltpu.VMEM((CH, D // 2), jnp.int32),
        pltpu.VMEM((CH, D), jnp.float32),
        pltpu.SemaphoreType.DMA,
    ],
)


ROWS_BLK = 1000
GRID = N // ROWS_BLK


def _gemm0_body(x_ref, w_ref, y_ref):
    y_ref[...] = jnp.dot(x_ref[...], w_ref[...].T,
                         preferred_element_type=jnp.float32
                         ).astype(jnp.bfloat16)


_gemm0 = pl.pallas_call(
    _gemm0_body,
    grid=(GRID,),
    in_specs=[
        pl.BlockSpec((ROWS_BLK, D), lambda i: (i, 0)),
        pl.BlockSpec((D, D), lambda i: (0, 0)),
    ],
    out_specs=pl.BlockSpec((ROWS_BLK, D), lambda i: (i, 0)),
    out_shape=jax.ShapeDtypeStruct((N, D), jnp.bfloat16),
)


def _gemm_mid_body(p_ref, w_ref, x_ref, y_ref):
    x = p_ref[0] + p_ref[1]
    x_ref[...] = x
    y_ref[...] = jnp.dot(x, w_ref[...].T,
                         preferred_element_type=jnp.float32
                         ).astype(jnp.bfloat16)


_gemm_mid = pl.pallas_call(
    _gemm_mid_body,
    grid=(GRID,),
    in_specs=[
        pl.BlockSpec((NC, ROWS_BLK, D), lambda i: (0, i, 0)),
        pl.BlockSpec((D, D), lambda i: (0, 0)),
    ],
    out_specs=[
        pl.BlockSpec((ROWS_BLK, D), lambda i: (i, 0)),
        pl.BlockSpec((ROWS_BLK, D), lambda i: (i, 0)),
    ],
    out_shape=[
        jax.ShapeDtypeStruct((N, D), jnp.float32),
        jax.ShapeDtypeStruct((N, D), jnp.bfloat16),
    ],
)


def _normed(x):
    nrm = jnp.sqrt(jnp.sum(x * x, axis=-1, keepdims=True))
    return x / jnp.maximum(nrm, 1e-12)


def _final_body(x0_ref, x1_ref, x2_ref, p_ref, o_ref):
    x3 = p_ref[0] + p_ref[1]
    o_ref[...] = 0.25 * (x0_ref[...] + _normed(x1_ref[...])
                         + _normed(x2_ref[...]) + _normed(x3))


_final = pl.pallas_call(
    _final_body,
    grid=(GRID,),
    in_specs=[
        pl.BlockSpec((ROWS_BLK, D), lambda i: (i, 0)),
        pl.BlockSpec((ROWS_BLK, D), lambda i: (i, 0)),
        pl.BlockSpec((ROWS_BLK, D), lambda i: (i, 0)),
        pl.BlockSpec((NC, ROWS_BLK, D), lambda i: (0, i, 0)),
    ],
    out_specs=pl.BlockSpec((ROWS_BLK, D), lambda i: (i, 0)),
    out_shape=jax.ShapeDtypeStruct((N, D), jnp.float32),
)


def _pack_pairs(y16):
    return jax.lax.bitcast_convert_type(
        y16.reshape(N, D // 2, 2), jnp.int32)


def kernel(adjacency_row, adjacency_col, adjacency_values, embedding, weights):
    pad = EPAD - E
    row3 = jnp.concatenate(
        [adjacency_row, jnp.zeros((pad,), jnp.int32)]).reshape(NW, NCH, CH)
    col3 = jnp.concatenate(
        [adjacency_col, jnp.zeros((pad,), jnp.int32)]).reshape(NW, NCH, CH)
    val3 = jnp.concatenate(
        [adjacency_values, jnp.zeros((pad,), jnp.float32)]).reshape(
            NW, NCH, CH)
    perm = jnp.asarray(_PERM)
    wp = weights[:, perm, :]

    y0 = _gemm0(embedding, wp[0])
    p1 = _spmm(_pack_pairs(y0), row3, col3, val3)
    x1, y1 = _gemm_mid(p1, wp[1])
    p2 = _spmm(_pack_pairs(y1), row3, col3, val3)
    x2, y2 = _gemm_mid(p2, wp[2])
    p3 = _spmm(_pack_pairs(y2), row3, col3, val3)
    return _final(embedding, x1, x2, p3)


# speedup vs baseline: 1.4934x; 1.0149x over previous
import numpy as np

import jax
import jax.numpy as jnp
from jax import lax
from jax.experimental import pallas as pl
from jax.experimental.pallas import tpu as pltpu
from jax.experimental.pallas import tpu_sc as plsc

N = 10000
E = 320000
D = 128
NC = 2
NS = 16
NW = NC * NS
CH = 64
NCH = 158
EPAD = NW * NCH * CH
ACC_N = 10240
RPT = ACC_N // NS

_PERM = np.empty((D,), dtype=np.int32)
for _g in range(D // 32):
    for _i in range(16):
        _PERM[32 * _g + 2 * _i] = 32 * _g + _i
        _PERM[32 * _g + 2 * _i + 1] = 32 * _g + 16 + _i

_MESH = plsc.VectorSubcoreMesh(core_axis_name="c", subcore_axis_name="s")


def _spmm_body(y_hbm, row_hbm, col_hbm, val_hbm, out_hbm,
               acc, rowb, colb, valb, gb0, gb1, sbuf, gs0, gs1):
    c = lax.axis_index("c")
    s = lax.axis_index("s")
    wid = c * NS + s

    pltpu.async_copy(row_hbm.at[wid], rowb, gs0)
    pltpu.async_copy(col_hbm.at[wid], colb, gs0)
    pltpu.async_copy(val_hbm.at[wid], valb, gs0)

    zero = jnp.zeros((16,), jnp.float32)

    def zb(i, carry):
        for j in range(8):
            sbuf[i, pl.ds(j * 16, 16)] = zero
        return carry

    lax.fori_loop(0, CH, zb, 0)
    for t in range(-(-RPT // CH)):
        off = min(t * CH, RPT - CH)
        pltpu.sync_copy(sbuf, acc.at[pl.ds(s * RPT + off, CH)])
    pltpu.make_async_copy(row_hbm.at[wid], rowb, gs0).wait()
    pltpu.make_async_copy(col_hbm.at[wid], colb, gs0).wait()
    pltpu.make_async_copy(val_hbm.at[wid], valb, gs0).wait()
    plsc.subcore_barrier()

    gbufs = (gb0, gb1)
    gsems = (gs0, gs1)
    pltpu.async_copy(y_hbm.at[colb.at[0]], gb0, gsems[0])

    def scale_chunk(gb, k):
        @plsc.parallel_loop(0, CH // 16)
        def edge_group(g):
            vvec = valb[k, pl.ds(g * 16, 16)]
            for l in range(16):
                v = vvec[l]
                e = g * 16 + l
                for j in range(D // 32):
                    u = gb[e, pl.ds(j * 16, 16)]
                    lo = plsc.bitcast(u << jnp.int32(16), jnp.float32)
                    hi = plsc.bitcast(u & jnp.int32(-65536), jnp.float32)
                    sbuf[e, pl.ds(j * 32, 16)] = lo * v
                    sbuf[e, pl.ds(j * 32 + 16, 16)] = hi * v

    def chunk_pair(m, carry):
        for b in range(2):
            k = 2 * m + b
            pltpu.make_async_copy(y_hbm.at[pl.ds(0, CH)], gbufs[b],
                                  gsems[b]).wait()

            @pl.when(k < NCH - 1)
            def _():
                pltpu.async_copy(y_hbm.at[colb.at[k + 1]], gbufs[1 - b],
                                 gsems[1 - b])

            scale_chunk(gbufs[b], k)
            pltpu.sync_copy(sbuf, acc.at[rowb.at[k]], add=True)
        return carry

    lax.fori_loop(0, NCH // 2, chunk_pair, 0)
    plsc.subcore_barrier()

    pltpu.sync_copy(acc.at[pl.ds(s * RPT, RPT)],
                    out_hbm.at[c, pl.ds(s * RPT, RPT)])


_spmm = pl.kernel(
    _spmm_body,
    out_type=jax.ShapeDtypeStruct((NC, ACC_N, D), jnp.float32),
    mesh=_MESH,
    compiler_params=pltpu.CompilerParams(use_tc_tiling_on_sc=False,
                                         needs_layout_passes=False),
    scratch_types=[
        pltpu.VMEM_SHARED((ACC_N, D), jnp.float32),
        pltpu.VMEM((NCH, CH), jnp.int32),
        pltpu.VMEM((NCH, CH), jnp.int32),
        pltpu.VMEM((NCH, CH), jnp.float32),
        pltpu.VMEM((CH, D // 2), jnp.int32),
        pltpu.VMEM((CH, D // 2), jnp.int32),
        pltpu.VMEM((CH, D), jnp.float32),
        pltpu.SemaphoreType.DMA,
        pltpu.SemaphoreType.DMA,
    ],
)


ROWS_BLK = 1000
GRID = N // ROWS_BLK


def _gemm0_body(x_ref, w_ref, y_ref):
    y_ref[...] = jnp.dot(x_ref[...], w_ref[...].T,
                         preferred_element_type=jnp.float32
                         ).astype(jnp.bfloat16)


_gemm0 = pl.pallas_call(
    _gemm0_body,
    grid=(GRID,),
    in_specs=[
        pl.BlockSpec((ROWS_BLK, D), lambda i: (i, 0)),
        pl.BlockSpec((D, D), lambda i: (0, 0)),
    ],
    out_specs=pl.BlockSpec((ROWS_BLK, D), lambda i: (i, 0)),
    out_shape=jax.ShapeDtypeStruct((N, D), jnp.bfloat16),
)


def _gemm_mid_body(p_ref, w_ref, x_ref, y_ref):
    x = p_ref[0] + p_ref[1]
    x_ref[...] = x
    y_ref[...] = jnp.dot(x, w_ref[...].T,
                         preferred_element_type=jnp.float32
                         ).astype(jnp.bfloat16)


_gemm_mid = pl.pallas_call(
    _gemm_mid_body,
    grid=(GRID,),
    in_specs=[
        pl.BlockSpec((NC, ROWS_BLK, D), lambda i: (0, i, 0)),
        pl.BlockSpec((D, D), lambda i: (0, 0)),
    ],
    out_specs=[
        pl.BlockSpec((ROWS_BLK, D), lambda i: (i, 0)),
        pl.BlockSpec((ROWS_BLK, D), lambda i: (i, 0)),
    ],
    out_shape=[
        jax.ShapeDtypeStruct((N, D), jnp.float32),
        jax.ShapeDtypeStruct((N, D), jnp.bfloat16),
    ],
)


def _normed(x):
    nrm = jnp.sqrt(jnp.sum(x * x, axis=-1, keepdims=True))
    return x / jnp.maximum(nrm, 1e-12)


def _final_body(x0_ref, x1_ref, x2_ref, p_ref, o_ref):
    x3 = p_ref[0] + p_ref[1]
    o_ref[...] = 0.25 * (x0_ref[...] + _normed(x1_ref[...])
                         + _normed(x2_ref[...]) + _normed(x3))


_final = pl.pallas_call(
    _final_body,
    grid=(GRID,),
    in_specs=[
        pl.BlockSpec((ROWS_BLK, D), lambda i: (i, 0)),
        pl.BlockSpec((ROWS_BLK, D), lambda i: (i, 0)),
        pl.BlockSpec((ROWS_BLK, D), lambda i: (i, 0)),
        pl.BlockSpec((NC, ROWS_BLK, D), lambda i: (0, i, 0)),
    ],
    out_specs=pl.BlockSpec((ROWS_BLK, D), lambda i: (i, 0)),
    out_shape=jax.ShapeDtypeStruct((N, D), jnp.float32),
)


def _pack_pairs(y16):
    return jax.lax.bitcast_convert_type(
        y16.reshape(N, D // 2, 2), jnp.int32)


def kernel(adjacency_row, adjacency_col, adjacency_values, embedding, weights):
    pad = EPAD - E
    row3 = jnp.concatenate(
        [adjacency_row, jnp.zeros((pad,), jnp.int32)]).reshape(NW, NCH, CH)
    col3 = jnp.concatenate(
        [adjacency_col, jnp.zeros((pad,), jnp.int32)]).reshape(NW, NCH, CH)
    val3 = jnp.concatenate(
        [adjacency_values, jnp.zeros((pad,), jnp.float32)]).reshape(
            NW, NCH, CH)
    perm = jnp.asarray(_PERM)
    wp = weights[:, perm, :]

    y0 = _gemm0(embedding, wp[0])
    p1 = _spmm(_pack_pairs(y0), row3, col3, val3)
    x1, y1 = _gemm_mid(p1, wp[1])
    p2 = _spmm(_pack_pairs(y1), row3, col3, val3)
    x2, y2 = _gemm_mid(p2, wp[2])
    p3 = _spmm(_pack_pairs(y2), row3, col3, val3)
    return _final(embedding, x1, x2, p3)
